# baseline (speedup 1.0000x reference)
import functools

import jax
import jax.numpy as jnp
from jax import lax
from jax.experimental import pallas as pl
from jax.experimental.pallas import tpu as pltpu
from jax.experimental.pallas import tpu_sc as plsc

T, D, INTER, E, K = 2048, 1024, 512, 8, 2
P = T * K
NC, NS, L = 2, 16, 16
NW = NC * NS
PPW = P // NW
CH = 32
NCH = PPW // CH
TCH = CH // K
BM = 256
NB = P // BM
NITEM = NB + E - 1
NSLOT = 32
GRID = E + NITEM

_mesh = plsc.VectorSubcoreMesh(core_axis_name="c", subcore_axis_name="s")


RCH = 128
NRCH = T // RCH


def _router_body(x_ref, wgr_ref, wur_ref, sb_ref, pp_ref, wp_ref, offs_ref,
                 xb_ref, rank_ref):
    xs = x_ref[...]
    g = jnp.dot(xs, wgr_ref[...].T, preferred_element_type=jnp.float32)
    u = jnp.dot(xs, wur_ref[...].T, preferred_element_type=jnp.float32)
    s = jnp.abs(u * (g * jax.nn.sigmoid(g)))
    s = jax.nn.softmax(s, axis=-1)
    scale = sb_ref[0:1, :]
    bias = sb_ref[1:2, :]
    sbias = s + bias
    iota = lax.broadcasted_iota(jnp.int32, s.shape, 1)
    m1 = jnp.max(sbias, axis=1, keepdims=True)
    i1 = jnp.min(jnp.where(sbias == m1, iota, E), axis=1, keepdims=True)
    oh1 = iota == i1
    sb2 = jnp.where(oh1, -jnp.inf, sbias)
    m2 = jnp.max(sb2, axis=1, keepdims=True)
    i2 = jnp.min(jnp.where((sb2 == m2) & (~oh1), iota, E), axis=1,
                 keepdims=True)
    oh2 = iota == i2
    wsc = 1.0 + s * scale
    w1 = jnp.sum(jnp.where(oh1, wsc, 0.0), axis=1, keepdims=True)
    w2 = jnp.sum(jnp.where(oh2, wsc, 0.0), axis=1, keepdims=True)
    wp_ref[...] = jnp.concatenate([w1, w2], axis=1)
    xb_ref[...] = xs.astype(jnp.bfloat16)

    A = jnp.where(oh1 | oh2, 1.0, 0.0).astype(jnp.bfloat16)
    ir = lax.broadcasted_iota(jnp.int32, (RCH, RCH), 0)
    ic = lax.broadcasted_iota(jnp.int32, (RCH, RCH), 1)
    Ltri = jnp.where(ir > ic, 1.0, 0.0).astype(jnp.bfloat16)
    cum = jnp.zeros((1, E), jnp.float32)
    for c in range(NRCH):
        Ac = A[c * RCH:(c + 1) * RCH, :]
        rank_c = jnp.dot(Ltri, Ac, preferred_element_type=jnp.float32)
        rank_ref[c * RCH:(c + 1) * RCH, :] = rank_c + cum
        cum = cum + rank_c[RCH - 1:RCH, :] + Ac[RCH - 1:RCH, :].astype(
            jnp.float32)
    lane16 = lax.broadcasted_iota(jnp.int32, (1, 2 * E), 1)
    off16 = jnp.zeros((1, 2 * E), jnp.float32)
    off8 = jnp.zeros((1, E), jnp.float32)
    for e in range(E):
        tot_e = cum[0:1, e:e + 1]
        off16 = off16 + jnp.where(lane16 > e, tot_e, 0.0)
        eiota = lax.broadcasted_iota(jnp.int32, (1, E), 1)
        off8 = off8 + jnp.where(eiota > e, tot_e, 0.0)
    offs_ref[...] = off16.astype(jnp.int32)
    pos_full = off8 + rank_ref[...]
    p1 = jnp.sum(jnp.where(oh1, pos_full, 0.0), axis=1, keepdims=True)
    p2 = jnp.sum(jnp.where(oh2, pos_full, 0.0), axis=1, keepdims=True)
    pp_ref[...] = jnp.concatenate([p1, p2], axis=1).astype(jnp.int32)


def _router(x, Wgate_r, Wup_r, sb):
    return pl.pallas_call(
        _router_body,
        out_shape=(
            jax.ShapeDtypeStruct((T, K), jnp.int32),
            jax.ShapeDtypeStruct((T, K), jnp.float32),
            jax.ShapeDtypeStruct((1, 2 * E), jnp.int32),
            jax.ShapeDtypeStruct((T, D), jnp.bfloat16),
        ),
        scratch_shapes=[pltpu.VMEM((T, E), jnp.float32)],
    )(x, Wgate_r, Wup_r, sb)


def _take16(v, idx):
    dnums = lax.GatherDimensionNumbers(
        offset_dims=(), collapsed_slice_dims=(0,), start_index_map=(0,))
    return lax.gather(v, idx[:, None], dnums, slice_sizes=(1,),
                      mode=lax.GatherScatterMode.PROMISE_IN_BOUNDS)


def _lane():
    return lax.broadcasted_iota(jnp.int32, (L,), 0)


def _allsum(v):
    lane = _lane()
    for d in (1, 2, 4, 8):
        v = v + _take16(v, lane ^ d)
    return v


def _excscan(v):
    lane = _lane()
    s = v
    for d in (1, 2, 4, 8):
        sh = _take16(s, jnp.maximum(lane - d, 0))
        s = s + jnp.where(lane >= d, sh, 0)
    return s - v


def _splat(x):
    return jnp.full((L,), x, jnp.int32)


BCH = 64
NBCH = PPW // BCH
D2 = D // 2


@functools.partial(
    pl.kernel, mesh=_mesh,
    out_type=(
        jax.ShapeDtypeStruct((P, D2), jnp.int32),
        jax.ShapeDtypeStruct((P,), jnp.float32),
        jax.ShapeDtypeStruct((5, NSLOT), jnp.int32),
    ),
    scratch_types=[
        pltpu.VMEM((NBCH, BCH), jnp.int32),
        pltpu.VMEM((NBCH, BCH), jnp.int32),
        pltpu.VMEM((NBCH, BCH), jnp.float32),
        pltpu.VMEM((2, BCH, D2), jnp.int32),
        pltpu.VMEM((L,), jnp.int32),
        pltpu.VMEM((5, NSLOT), jnp.int32),
        pltpu.SemaphoreType.DMA,
        pltpu.SemaphoreType.DMA,
    ],
)
def _sc_dispatch(pos_hbm, w_hbm, x_hbm, offs_hbm, xs_hbm, ws_hbm, sched_hbm,
                 pos_v, tok_v, w_v, rows_v, off_v, sched_v, sem, sem2):
    wid = lax.axis_index("s") * NC + lax.axis_index("c")
    base = wid * PPW

    pltpu.sync_copy(pos_hbm.at[pl.ds(wid * NBCH, NBCH)], pos_v)
    pltpu.sync_copy(w_hbm.at[pl.ds(wid * NBCH, NBCH)], w_v)

    for c in range(NBCH):
        for h in range(BCH // L):
            jj = _lane() + (base + c * BCH + h * L)
            tok_v[c, pl.ds(h * L, L)] = lax.shift_right_logical(jj, 1)

    for c in range(NBCH):
        pltpu.async_copy(w_v.at[c], ws_hbm.at[pos_v.at[c]], sem).wait()

    pltpu.async_copy(x_hbm.at[tok_v.at[0]], rows_v.at[0], sem).wait()
    for c in range(NBCH):
        if c + 1 < NBCH:
            nxt = pltpu.async_copy(x_hbm.at[tok_v.at[c + 1]],
                                   rows_v.at[(c + 1) % 2], sem)
        pltpu.async_copy(rows_v.at[c % 2], xs_hbm.at[pos_v.at[c]], sem2).wait()
        if c + 1 < NBCH:
            nxt.wait()

    @pl.when(wid == 0)
    def _sched():
        lane = _lane()
        pltpu.sync_copy(offs_hbm, off_v)
        off_lane = off_v[...]
        rstart = lane * BM
        rend = rstart + (BM - 1)
        fe = jnp.zeros((L,), jnp.int32)
        le = jnp.zeros((L,), jnp.int32)
        for e in range(E):
            off_e1 = _splat(off_lane[e + 1])
            fe = fe + jnp.where(off_e1 <= rstart, 1, 0)
            le = le + jnp.where(off_e1 <= rend, 1, 0)
        nit = le - fe + 1
        Citems = _excscan(nit)
        total_items = Citems[NB - 1] + nit[NB - 1]
        for sv in range(NSLOT // L):
            wv = lane + sv * L
            b_ws = jnp.zeros((L,), jnp.int32) - 1
            for b in range(NB):
                b_ws = b_ws + jnp.where(wv >= _splat(Citems[b]), 1, 0)
            b_ws = jnp.clip(b_ws, 0, NB - 1)
            C_at = _take16(Citems, b_ws)
            fe_at = _take16(fe, b_ws)
            e_ws = fe_at + (wv - C_at)
            e_c = jnp.clip(e_ws, 0, E - 1)
            lo = jnp.maximum(_take16(off_lane, e_c), b_ws * BM)
            hi = jnp.minimum(_take16(off_lane, e_c + 1), (b_ws + 1) * BM)
            valid = wv < _splat(total_items)
            sched_v[0, pl.ds(sv * L, L)] = jnp.where(valid, b_ws, NB - 1)
            sched_v[1, pl.ds(sv * L, L)] = jnp.where(valid, e_c, 0)
            sched_v[2, pl.ds(sv * L, L)] = jnp.where(valid, lo, 0)
            sched_v[3, pl.ds(sv * L, L)] = jnp.where(valid, hi, 0)
            sched_v[4, pl.ds(sv * L, L)] = jnp.where(
                valid & (e_ws == fe_at), 1, 0)
        pltpu.sync_copy(sched_v, sched_hbm)


def _ffn_body(sched_ref, xs_ref, ws_ref, wg_ref, wu_ref, wd_ref, ys_ref,
              wgb_ref, wub_ref, wdb_ref):
    i = pl.program_id(0)

    @pl.when(i < E)
    def _cast():
        wgb_ref[i] = wg_ref[0].astype(jnp.bfloat16)
        wub_ref[i] = wu_ref[0].astype(jnp.bfloat16)
        wdb_ref[i] = wd_ref[0].astype(jnp.bfloat16)

    @pl.when(i >= E)
    def _item():
        w = i - E
        blk = sched_ref[0, w]
        e = sched_ref[1, w]
        lo = sched_ref[2, w]
        hi = sched_ref[3, w]
        first = sched_ref[4, w]
        xb = xs_ref[...]
        dn = (((1,), (1,)), ((), ()))
        g = lax.dot_general(xb, wgb_ref[e], dn,
                            preferred_element_type=jnp.float32)
        u = lax.dot_general(xb, wub_ref[e], dn,
                            preferred_element_type=jnp.float32)
        r = blk * BM + lax.broadcasted_iota(jnp.int32, (BM, 1), 0)
        m = (r >= lo) & (r < hi)
        wrow = jnp.where(m, ws_ref[...], 0.0)
        h = (g * jax.nn.sigmoid(g) * u * wrow).astype(jnp.bfloat16)
        o = lax.dot_general(h, wdb_ref[e], dn,
                            preferred_element_type=jnp.float32)

        @pl.when(first == 1)
        def _init():
            ys_ref[...] = o

        @pl.when(first == 0)
        def _acc():
            ys_ref[...] += o


def _ffn(sched, xs, ws, Wg, Wu, Wd):
    def wmap(i, s):
        return (jnp.minimum(i, E - 1), 0, 0)

    def bmap(i, s):
        return (s[0, jnp.maximum(i - E, 0)], 0)

    grid_spec = pltpu.PrefetchScalarGridSpec(
        num_scalar_prefetch=1,
        grid=(GRID,),
        in_specs=[
            pl.BlockSpec((BM, D), bmap),
            pl.BlockSpec((BM, 1), bmap),
            pl.BlockSpec((1, INTER, D), wmap),
            pl.BlockSpec((1, INTER, D), wmap),
            pl.BlockSpec((1, D, INTER), wmap),
        ],
        out_specs=pl.BlockSpec((BM, D), bmap),
        scratch_shapes=[
            pltpu.VMEM((E, INTER, D), jnp.bfloat16),
            pltpu.VMEM((E, INTER, D), jnp.bfloat16),
            pltpu.VMEM((E, D, INTER), jnp.bfloat16),
        ],
    )
    return pl.pallas_call(
        _ffn_body,
        grid_spec=grid_spec,
        out_shape=jax.ShapeDtypeStruct((P, D), jnp.float32),
        compiler_params=pltpu.CompilerParams(
            dimension_semantics=("arbitrary",),
        ),
    )(sched, xs, ws, Wg, Wu, Wd)


BTCH = BCH // K


@functools.partial(
    pl.kernel, mesh=_mesh,
    out_type=jax.ShapeDtypeStruct((T, D), jnp.float32),
    scratch_types=[
        pltpu.VMEM((NCH, CH), jnp.int32),
        pltpu.VMEM((2, CH, D), jnp.float32),
        pltpu.VMEM((TCH, D), jnp.float32),
        pltpu.SemaphoreType.DMA,
    ],
)
def _sc_combine(ys_hbm, posf_hbm, y_hbm, pos_v, rows_v, out_v, sem):
    wid = lax.axis_index("s") * NC + lax.axis_index("c")
    pltpu.sync_copy(posf_hbm.at[pl.ds(wid * NCH, NCH)], pos_v)

    pltpu.async_copy(ys_hbm.at[pos_v.at[0]], rows_v.at[0], sem).wait()
    for c in range(NCH):
        if c + 1 < NCH:
            nxt = pltpu.async_copy(ys_hbm.at[pos_v.at[c + 1]],
                                   rows_v.at[(c + 1) % 2], sem)

        def body(i, _, c=c):
            for k in range(D // L):
                a = rows_v[c % 2, 2 * i, pl.ds(k * L, L)]
                b = rows_v[c % 2, 2 * i + 1, pl.ds(k * L, L)]
                out_v[i, pl.ds(k * L, L)] = a + b
            return _

        lax.fori_loop(0, TCH, body, 0)
        pltpu.sync_copy(out_v, y_hbm.at[pl.ds(wid * (PPW // K) + c * TCH,
                                              TCH)])
        if c + 1 < NCH:
            nxt.wait()


@jax.jit
def kernel(x, Wgate_r, Wup_r, extra_scale, extra_bias, Wg, Wu, Wd):
    sb = jnp.stack([extra_scale, extra_bias])
    pp, wp, offs, xb = _router(x, Wgate_r, Wup_r, sb)
    pos2d = pp.reshape(P // BCH, BCH)
    w2d = wp.reshape(P // BCH, BCH)
    xbi = lax.bitcast_convert_type(xb.reshape(T, D2, 2), jnp.int32)
    xsi, ws, sched = _sc_dispatch(pos2d, w2d, xbi, offs.reshape(2 * E))
    xs = lax.bitcast_convert_type(xsi, jnp.bfloat16).reshape(P, D)
    ys = _ffn(sched, xs, ws.reshape(P, 1), Wg, Wu, Wd)
    return _sc_combine(ys, pp.reshape(P // CH, CH))


# speedup vs baseline: 1.7183x; 1.7183x over previous
import functools

import jax
import jax.numpy as jnp
from jax import lax
from jax.experimental import pallas as pl
from jax.experimental.pallas import tpu as pltpu
from jax.experimental.pallas import tpu_sc as plsc

T, D, INTER, E, K = 2048, 1024, 512, 8, 2
P = T * K
NC, NS, L = 2, 16, 16
NW = NC * NS
PPW = P // NW
CH = 32
NCH = PPW // CH
TCH = CH // K
BM = 256
NB = P // BM
NITEM = NB + E - 1
NSLOT = 32
GRID = E + NITEM

_mesh = plsc.VectorSubcoreMesh(core_axis_name="c", subcore_axis_name="s")


RCH = 128
NRCH = T // RCH


def _router_body(x_ref, wgr_ref, wur_ref, sb_ref, pp_ref, wp_ref, offs_ref,
                 rank_ref):
    xs = x_ref[...]
    g = jnp.dot(xs, wgr_ref[...].T, preferred_element_type=jnp.float32)
    u = jnp.dot(xs, wur_ref[...].T, preferred_element_type=jnp.float32)
    s = jnp.abs(u * (g * jax.nn.sigmoid(g)))
    s = jax.nn.softmax(s, axis=-1)
    scale = sb_ref[0:1, :]
    bias = sb_ref[1:2, :]
    sbias = s + bias
    iota = lax.broadcasted_iota(jnp.int32, s.shape, 1)
    m1 = jnp.max(sbias, axis=1, keepdims=True)
    i1 = jnp.min(jnp.where(sbias == m1, iota, E), axis=1, keepdims=True)
    oh1 = iota == i1
    sb2 = jnp.where(oh1, -jnp.inf, sbias)
    m2 = jnp.max(sb2, axis=1, keepdims=True)
    i2 = jnp.min(jnp.where((sb2 == m2) & (~oh1), iota, E), axis=1,
                 keepdims=True)
    oh2 = iota == i2
    wsc = 1.0 + s * scale
    w1 = jnp.sum(jnp.where(oh1, wsc, 0.0), axis=1, keepdims=True)
    w2 = jnp.sum(jnp.where(oh2, wsc, 0.0), axis=1, keepdims=True)
    wp_ref[...] = jnp.concatenate([w1, w2], axis=1)

    A = jnp.where(oh1 | oh2, 1.0, 0.0).astype(jnp.bfloat16)
    ir = lax.broadcasted_iota(jnp.int32, (RCH, RCH), 0)
    ic = lax.broadcasted_iota(jnp.int32, (RCH, RCH), 1)
    Ltri = jnp.where(ir > ic, 1.0, 0.0).astype(jnp.bfloat16)
    cum = jnp.zeros((1, E), jnp.float32)
    for c in range(NRCH):
        Ac = A[c * RCH:(c + 1) * RCH, :]
        rank_c = jnp.dot(Ltri, Ac, preferred_element_type=jnp.float32)
        rank_ref[c * RCH:(c + 1) * RCH, :] = rank_c + cum
        cum = cum + rank_c[RCH - 1:RCH, :] + Ac[RCH - 1:RCH, :].astype(
            jnp.float32)
    lane16 = lax.broadcasted_iota(jnp.int32, (1, 2 * E), 1)
    off16 = jnp.zeros((1, 2 * E), jnp.float32)
    off8 = jnp.zeros((1, E), jnp.float32)
    for e in range(E):
        tot_e = cum[0:1, e:e + 1]
        off16 = off16 + jnp.where(lane16 > e, tot_e, 0.0)
        eiota = lax.broadcasted_iota(jnp.int32, (1, E), 1)
        off8 = off8 + jnp.where(eiota > e, tot_e, 0.0)
    offs_ref[...] = off16.astype(jnp.int32)
    pos_full = off8 + rank_ref[...]
    p1 = jnp.sum(jnp.where(oh1, pos_full, 0.0), axis=1, keepdims=True)
    p2 = jnp.sum(jnp.where(oh2, pos_full, 0.0), axis=1, keepdims=True)
    pp_ref[...] = jnp.concatenate([p1, p2], axis=1).astype(jnp.int32)


def _router(x, Wgate_r, Wup_r, sb):
    return pl.pallas_call(
        _router_body,
        out_shape=(
            jax.ShapeDtypeStruct((T, K), jnp.int32),
            jax.ShapeDtypeStruct((T, K), jnp.float32),
            jax.ShapeDtypeStruct((1, 2 * E), jnp.int32),
        ),
        scratch_shapes=[pltpu.VMEM((T, E), jnp.float32)],
    )(x, Wgate_r, Wup_r, sb)


def _take16(v, idx):
    dnums = lax.GatherDimensionNumbers(
        offset_dims=(), collapsed_slice_dims=(0,), start_index_map=(0,))
    return lax.gather(v, idx[:, None], dnums, slice_sizes=(1,),
                      mode=lax.GatherScatterMode.PROMISE_IN_BOUNDS)


def _lane():
    return lax.broadcasted_iota(jnp.int32, (L,), 0)


def _allsum(v):
    lane = _lane()
    for d in (1, 2, 4, 8):
        v = v + _take16(v, lane ^ d)
    return v


def _excscan(v):
    lane = _lane()
    s = v
    for d in (1, 2, 4, 8):
        sh = _take16(s, jnp.maximum(lane - d, 0))
        s = s + jnp.where(lane >= d, sh, 0)
    return s - v


def _splat(x):
    return jnp.full((L,), x, jnp.int32)


@functools.partial(
    pl.kernel, mesh=_mesh,
    out_type=(
        jax.ShapeDtypeStruct((P, D), jnp.float32),
        jax.ShapeDtypeStruct((P,), jnp.float32),
        jax.ShapeDtypeStruct((5, NSLOT), jnp.int32),
    ),
    scratch_types=[
        pltpu.VMEM((NCH, CH), jnp.int32),
        pltpu.VMEM((NCH, CH), jnp.int32),
        pltpu.VMEM((NCH, CH), jnp.float32),
        pltpu.VMEM((2, CH, D), jnp.float32),
        pltpu.VMEM((L,), jnp.int32),
        pltpu.VMEM((5, NSLOT), jnp.int32),
        pltpu.SemaphoreType.DMA,
        pltpu.SemaphoreType.DMA,
    ],
)
def _sc_dispatch(pos_hbm, w_hbm, x_hbm, offs_hbm, xs_hbm, ws_hbm, sched_hbm,
                 pos_v, tok_v, w_v, rows_v, off_v, sched_v, sem, sem2):
    wid = lax.axis_index("s") * NC + lax.axis_index("c")
    base = wid * PPW

    pltpu.sync_copy(pos_hbm.at[pl.ds(wid * NCH, NCH)], pos_v)
    pltpu.sync_copy(w_hbm.at[pl.ds(wid * NCH, NCH)], w_v)

    for c in range(NCH):
        for h in range(CH // L):
            jj = _lane() + (base + c * CH + h * L)
            tok_v[c, pl.ds(h * L, L)] = lax.shift_right_logical(jj, 1)

    for c in range(NCH):
        pltpu.async_copy(w_v.at[c], ws_hbm.at[pos_v.at[c]], sem).wait()

    pltpu.async_copy(x_hbm.at[tok_v.at[0]], rows_v.at[0], sem).wait()
    for c in range(NCH):
        if c + 1 < NCH:
            nxt = pltpu.async_copy(x_hbm.at[tok_v.at[c + 1]],
                                   rows_v.at[(c + 1) % 2], sem)
        pltpu.async_copy(rows_v.at[c % 2], xs_hbm.at[pos_v.at[c]], sem2).wait()
        if c + 1 < NCH:
            nxt.wait()

    @pl.when(wid == 0)
    def _sched():
        lane = _lane()
        pltpu.sync_copy(offs_hbm, off_v)
        off_lane = off_v[...]
        rstart = lane * BM
        rend = rstart + (BM - 1)
        fe = jnp.zeros((L,), jnp.int32)
        le = jnp.zeros((L,), jnp.int32)
        for e in range(E):
            off_e1 = _splat(off_lane[e + 1])
            fe = fe + jnp.where(off_e1 <= rstart, 1, 0)
            le = le + jnp.where(off_e1 <= rend, 1, 0)
        nit = le - fe + 1
        Citems = _excscan(nit)
        total_items = Citems[NB - 1] + nit[NB - 1]
        for sv in range(NSLOT // L):
            wv = lane + sv * L
            b_ws = jnp.zeros((L,), jnp.int32) - 1
            for b in range(NB):
                b_ws = b_ws + jnp.where(wv >= _splat(Citems[b]), 1, 0)
            b_ws = jnp.clip(b_ws, 0, NB - 1)
            C_at = _take16(Citems, b_ws)
            fe_at = _take16(fe, b_ws)
            e_ws = fe_at + (wv - C_at)
            e_c = jnp.clip(e_ws, 0, E - 1)
            lo = jnp.maximum(_take16(off_lane, e_c), b_ws * BM)
            hi = jnp.minimum(_take16(off_lane, e_c + 1), (b_ws + 1) * BM)
            valid = wv < _splat(total_items)
            sched_v[0, pl.ds(sv * L, L)] = jnp.where(valid, b_ws, NB - 1)
            sched_v[1, pl.ds(sv * L, L)] = jnp.where(valid, e_c, 0)
            sched_v[2, pl.ds(sv * L, L)] = jnp.where(valid, lo, 0)
            sched_v[3, pl.ds(sv * L, L)] = jnp.where(valid, hi, 0)
            sched_v[4, pl.ds(sv * L, L)] = jnp.where(
                valid & (e_ws == fe_at), 1, 0)
        pltpu.sync_copy(sched_v, sched_hbm)


def _ffn_body(sched_ref, xs_ref, ws_ref, wg_ref, wu_ref, wd_ref, ys_ref,
              wgb_ref, wub_ref, wdb_ref):
    i = pl.program_id(0)

    @pl.when(i < E)
    def _cast():
        wgb_ref[i] = wg_ref[0].astype(jnp.bfloat16)
        wub_ref[i] = wu_ref[0].astype(jnp.bfloat16)
        wdb_ref[i] = wd_ref[0].astype(jnp.bfloat16)

    @pl.when(i >= E)
    def _item():
        w = i - E
        blk = sched_ref[0, w]
        e = sched_ref[1, w]
        lo = sched_ref[2, w]
        hi = sched_ref[3, w]
        first = sched_ref[4, w]
        xb = xs_ref[...].astype(jnp.bfloat16)
        dn = (((1,), (1,)), ((), ()))
        g = lax.dot_general(xb, wgb_ref[e], dn,
                            preferred_element_type=jnp.float32)
        u = lax.dot_general(xb, wub_ref[e], dn,
                            preferred_element_type=jnp.float32)
        r = blk * BM + lax.broadcasted_iota(jnp.int32, (BM, 1), 0)
        m = (r >= lo) & (r < hi)
        wrow = jnp.where(m, ws_ref[...], 0.0)
        h = (g * jax.nn.sigmoid(g) * u * wrow).astype(jnp.bfloat16)
        o = lax.dot_general(h, wdb_ref[e], dn,
                            preferred_element_type=jnp.float32)

        @pl.when(first == 1)
        def _init():
            ys_ref[...] = o

        @pl.when(first == 0)
        def _acc():
            ys_ref[...] += o


def _ffn(sched, xs, ws, Wg, Wu, Wd):
    def wmap(i, s):
        return (jnp.minimum(i, E - 1), 0, 0)

    def bmap(i, s):
        return (s[0, jnp.maximum(i - E, 0)], 0)

    grid_spec = pltpu.PrefetchScalarGridSpec(
        num_scalar_prefetch=1,
        grid=(GRID,),
        in_specs=[
            pl.BlockSpec((BM, D), bmap),
            pl.BlockSpec((BM, 1), bmap),
            pl.BlockSpec((1, INTER, D), wmap),
            pl.BlockSpec((1, INTER, D), wmap),
            pl.BlockSpec((1, D, INTER), wmap),
        ],
        out_specs=pl.BlockSpec((BM, D), bmap),
        scratch_shapes=[
            pltpu.VMEM((E, INTER, D), jnp.bfloat16),
            pltpu.VMEM((E, INTER, D), jnp.bfloat16),
            pltpu.VMEM((E, D, INTER), jnp.bfloat16),
        ],
    )
    return pl.pallas_call(
        _ffn_body,
        grid_spec=grid_spec,
        out_shape=jax.ShapeDtypeStruct((P, D), jnp.float32),
        compiler_params=pltpu.CompilerParams(
            dimension_semantics=("arbitrary",),
        ),
    )(sched, xs, ws, Wg, Wu, Wd)


@functools.partial(
    pl.kernel, mesh=_mesh,
    out_type=jax.ShapeDtypeStruct((T, D), jnp.float32),
    scratch_types=[
        pltpu.VMEM((NCH, CH), jnp.int32),
        pltpu.VMEM((2, CH, D), jnp.float32),
        pltpu.VMEM((TCH, D), jnp.float32),
        pltpu.SemaphoreType.DMA,
    ],
)
def _sc_combine(ys_hbm, posf_hbm, y_hbm, pos_v, rows_v, out_v, sem):
    wid = lax.axis_index("s") * NC + lax.axis_index("c")
    pltpu.sync_copy(posf_hbm.at[pl.ds(wid * NCH, NCH)], pos_v)

    pltpu.async_copy(ys_hbm.at[pos_v.at[0]], rows_v.at[0], sem).wait()
    for c in range(NCH):
        if c + 1 < NCH:
            nxt = pltpu.async_copy(ys_hbm.at[pos_v.at[c + 1]],
                                   rows_v.at[(c + 1) % 2], sem)

        def body(i, _, c=c):
            for k in range(D // L):
                a = rows_v[c % 2, 2 * i, pl.ds(k * L, L)]
                b = rows_v[c % 2, 2 * i + 1, pl.ds(k * L, L)]
                out_v[i, pl.ds(k * L, L)] = a + b
            return _

        lax.fori_loop(0, TCH, body, 0)
        pltpu.sync_copy(out_v, y_hbm.at[pl.ds(wid * (PPW // K) + c * TCH,
                                              TCH)])
        if c + 1 < NCH:
            nxt.wait()


@jax.jit
def kernel(x, Wgate_r, Wup_r, extra_scale, extra_bias, Wg, Wu, Wd):
    sb = jnp.stack([extra_scale, extra_bias])
    pp, wp, offs = _router(x, Wgate_r, Wup_r, sb)
    pos2d = pp.reshape(P // CH, CH)
    w2d = wp.reshape(P // CH, CH)
    xs, ws, sched = _sc_dispatch(pos2d, w2d, x, offs.reshape(2 * E))
    ys = _ffn(sched, xs, ws.reshape(P, 1), Wg, Wu, Wd)
    return _sc_combine(ys, pos2d)


# speedup vs baseline: 3.3181x; 1.9310x over previous
import functools

import jax
import jax.numpy as jnp
from jax.experimental import pallas as pl
from jax.experimental.pallas import tpu as pltpu


def _moe_body(x_ref, wgr_ref, wur_ref, sb_ref, wg_ref, wu_ref, wd_ref,
              y_ref, wfull_ref, xb_ref):
    e = pl.program_id(0)
    E = wfull_ref.shape[1]

    @pl.when(e == 0)
    def _router():
        xs = x_ref[...]
        xb_ref[...] = xs.astype(jnp.bfloat16)
        g = jnp.dot(xs, wgr_ref[...].T, preferred_element_type=jnp.float32)
        u = jnp.dot(xs, wur_ref[...].T, preferred_element_type=jnp.float32)
        s = jnp.abs(u * (g * jax.nn.sigmoid(g)))
        s = jax.nn.softmax(s, axis=-1)
        scale = sb_ref[0:1, :]
        bias = sb_ref[1:2, :]
        sbias = s + bias
        iota = jax.lax.broadcasted_iota(jnp.int32, s.shape, 1)
        m1 = jnp.max(sbias, axis=1, keepdims=True)
        i1 = jnp.min(jnp.where(sbias == m1, iota, E), axis=1, keepdims=True)
        oh1 = iota == i1
        sb2 = jnp.where(oh1, -jnp.inf, sbias)
        m2 = jnp.max(sb2, axis=1, keepdims=True)
        i2 = jnp.min(jnp.where((sb2 == m2) & (~oh1), iota, E), axis=1,
                     keepdims=True)
        sel = oh1 | (iota == i2)
        wfull_ref[...] = jnp.where(sel, 1.0 + s * scale, 0.0)

    xb = xb_ref[...]
    wg = wg_ref[0].astype(jnp.bfloat16)
    wu = wu_ref[0].astype(jnp.bfloat16)
    wd = wd_ref[0].astype(jnp.bfloat16)
    dn = (((1,), (1,)), ((), ()))
    g = jax.lax.dot_general(xb, wg, dn, preferred_element_type=jnp.float32)
    u = jax.lax.dot_general(xb, wu, dn, preferred_element_type=jnp.float32)
    iota = jax.lax.broadcasted_iota(jnp.int32, wfull_ref.shape, 1)
    w_col = jnp.sum(jnp.where(iota == e, wfull_ref[...], 0.0), axis=1,
                    keepdims=True)
    h = (g * jax.nn.sigmoid(g) * u * w_col).astype(jnp.bfloat16)
    o = jax.lax.dot_general(h, wd, dn, preferred_element_type=jnp.float32)

    @pl.when(e == 0)
    def _init():
        y_ref[...] = o

    @pl.when(e > 0)
    def _acc():
        y_ref[...] += o


@jax.jit
def kernel(x, Wgate_r, Wup_r, extra_scale, extra_bias, Wg, Wu, Wd):
    T, D = x.shape
    E, INTER, _ = Wg.shape
    sb = jnp.stack([extra_scale, extra_bias])
    grid = (E,)
    return pl.pallas_call(
        _moe_body,
        grid=grid,
        in_specs=[
            pl.BlockSpec((T, D), lambda e: (0, 0)),
            pl.BlockSpec((E, D), lambda e: (0, 0)),
            pl.BlockSpec((E, D), lambda e: (0, 0)),
            pl.BlockSpec((2, E), lambda e: (0, 0)),
            pl.BlockSpec((1, INTER, D), lambda e: (e, 0, 0)),
            pl.BlockSpec((1, INTER, D), lambda e: (e, 0, 0)),
            pl.BlockSpec((1, D, INTER), lambda e: (e, 0, 0)),
        ],
        out_specs=pl.BlockSpec((T, D), lambda e: (0, 0)),
        out_shape=jax.ShapeDtypeStruct((T, D), jnp.float32),
        scratch_shapes=[pltpu.VMEM((T, E), jnp.float32),
                        pltpu.VMEM((T, D), jnp.bfloat16)],
        compiler_params=pltpu.CompilerParams(
            dimension_semantics=("arbitrary",),
        ),
    )(x, Wgate_r, Wup_r, sb, Wg, Wu, Wd)
